# baseline (speedup 1.0000x reference)
import functools

import jax
import jax.numpy as jnp
from jax import lax
from jax.experimental import pallas as pl
from jax.experimental.pallas import tpu as pltpu
from jax.experimental.pallas import tpu_sc as plsc

VOCAB = 50304
HIDDEN = 1024
N_TOK = 4 * 2048
NC = 2
NS = 16
LANES = 16
NW = NC * NS
TOK_PER_W = N_TOK // NW
CHUNK = 32
N_CHUNK = TOK_PER_W // CHUNK
SLICES_PER_ROW = HIDDEN // LANES

_mesh = plsc.VectorSubcoreMesh(core_axis_name="c", subcore_axis_name="s")


@functools.partial(
    pl.kernel,
    mesh=_mesh,
    out_type=jax.ShapeDtypeStruct((N_TOK, HIDDEN), jnp.float32),
    scratch_types=[
        pltpu.VMEM((TOK_PER_W,), jnp.int32),
        pltpu.VMEM((TOK_PER_W,), jnp.int32),
        pltpu.VMEM((CHUNK, HIDDEN), jnp.float32),
        pltpu.VMEM((CHUNK, HIDDEN), jnp.float32),
        pltpu.SemaphoreType.DMA,
        pltpu.SemaphoreType.DMA,
    ],
)
def _embed_sum(ids_hbm, pos_hbm, wtab_hbm, ptab_hbm, out_hbm,
               ids_v, pids_v, wrows, prows, sem_w, sem_p):
    wid = lax.axis_index("s") * NC + lax.axis_index("c")
    base = wid * TOK_PER_W
    pltpu.sync_copy(ids_hbm.at[pl.ds(base, TOK_PER_W)], ids_v)
    pltpu.sync_copy(pos_hbm.at[pl.ds(base, TOK_PER_W)], pids_v)

    def chunk_body(ci, carry):
        off = ci * CHUNK
        cw = pltpu.async_copy(wtab_hbm.at[ids_v.at[pl.ds(off, CHUNK)]], wrows, sem_w)
        cp = pltpu.async_copy(ptab_hbm.at[pids_v.at[pl.ds(off, CHUNK)]], prows, sem_p)
        cw.wait()
        cp.wait()

        def row_body(r, carry2):
            def slice_body(j, carry3):
                sl = pl.ds(j * LANES, LANES)
                wrows[r, sl] = wrows[r, sl] + prows[r, sl]
                return carry3
            return lax.fori_loop(0, SLICES_PER_ROW, slice_body, carry2)

        lax.fori_loop(0, CHUNK, row_body, 0)
        pltpu.sync_copy(wrows, out_hbm.at[pl.ds(base + off, CHUNK)])
        return carry

    lax.fori_loop(0, N_CHUNK, chunk_body, 0)


def kernel(input_ids, position_ids, word_embeddings, position_embeddings):
    ids = input_ids.reshape(-1).astype(jnp.int32)
    pos = position_ids.reshape(-1).astype(jnp.int32)
    out = _embed_sum(ids, pos, word_embeddings, position_embeddings)
    return out.reshape(input_ids.shape[0], input_ids.shape[1], HIDDEN)


# speedup vs baseline: 1.6529x; 1.6529x over previous
import functools

import jax
import jax.numpy as jnp
from jax import lax
from jax.experimental import pallas as pl
from jax.experimental.pallas import tpu as pltpu
from jax.experimental.pallas import tpu_sc as plsc

VOCAB = 50304
HIDDEN = 1024
N_TOK = 4 * 2048
NC = 2
NS = 16
LANES = 16
NW = NC * NS
TOK_PER_W = N_TOK // NW
CHUNK = 16
N_CHUNK = TOK_PER_W // CHUNK
NBUF = 3
SLICES_PER_ROW = HIDDEN // LANES

_mesh = plsc.VectorSubcoreMesh(core_axis_name="c", subcore_axis_name="s")


@functools.partial(
    pl.kernel,
    mesh=_mesh,
    out_type=jax.ShapeDtypeStruct((N_TOK, HIDDEN), jnp.float32),
    scratch_types=[
        pltpu.VMEM((TOK_PER_W,), jnp.int32),
        pltpu.VMEM((TOK_PER_W,), jnp.int32),
        pltpu.VMEM((NBUF, CHUNK, HIDDEN), jnp.float32),
        pltpu.VMEM((NBUF, CHUNK, HIDDEN), jnp.float32),
    ]
    + [pltpu.SemaphoreType.DMA] * (2 * NBUF),
)
def _embed_sum(ids_hbm, pos_hbm, wtab_hbm, ptab_hbm, out_hbm,
               ids_v, pids_v, wbuf, pbuf, *sems):
    gsem = sems[:NBUF]
    ssem = sems[NBUF:]
    wid = lax.axis_index("s") * NC + lax.axis_index("c")
    base = wid * TOK_PER_W
    pltpu.sync_copy(ids_hbm.at[pl.ds(base, TOK_PER_W)], ids_v)
    pltpu.sync_copy(pos_hbm.at[pl.ds(base, TOK_PER_W)], pids_v)

    def start_g(ci):
        b = ci % NBUF
        idx = pl.ds(ci * CHUNK, CHUNK)
        cw = pltpu.async_copy(wtab_hbm.at[ids_v.at[idx]], wbuf.at[b], gsem[b])
        cp = pltpu.async_copy(ptab_hbm.at[pids_v.at[idx]], pbuf.at[b], gsem[b])
        return cw, cp

    def start_st(ci):
        b = ci % NBUF
        return pltpu.async_copy(
            wbuf.at[b], out_hbm.at[pl.ds(base + ci * CHUNK, CHUNK)], ssem[b])

    g_h = {0: start_g(0), 1: start_g(1)}
    st_h = {}
    for ci in range(N_CHUNK):
        b = ci % NBUF
        cw, cp = g_h.pop(ci)
        cw.wait()
        cp.wait()

        def row_body(r, carry, _b=b):
            for j in range(SLICES_PER_ROW):
                sl = pl.ds(j * LANES, LANES)
                wbuf[_b, r, sl] = wbuf[_b, r, sl] + pbuf[_b, r, sl]
            return carry

        lax.fori_loop(0, CHUNK, row_body, 0)

        if ci + 2 < N_CHUNK:
            if ci - 1 >= 0:
                st_h.pop(ci - 1).wait()
            g_h[ci + 2] = start_g(ci + 2)
        st_h[ci] = start_st(ci)
    for ci in sorted(st_h):
        st_h.pop(ci).wait()


def kernel(input_ids, position_ids, word_embeddings, position_embeddings):
    ids = input_ids.reshape(-1).astype(jnp.int32)
    pos = position_ids.reshape(-1).astype(jnp.int32)
    out = _embed_sum(ids, pos, word_embeddings, position_embeddings)
    return out.reshape(input_ids.shape[0], input_ids.shape[1], HIDDEN)


# speedup vs baseline: 1.8766x; 1.1353x over previous
import functools

import jax
import jax.numpy as jnp
from jax import lax
from jax.experimental import pallas as pl
from jax.experimental.pallas import tpu as pltpu
from jax.experimental.pallas import tpu_sc as plsc

VOCAB = 50304
HIDDEN = 1024
N_TOK = 4 * 2048
NC = 2
NS = 16
LANES = 16
NW = NC * NS
TOK_PER_W = N_TOK // NW
CHUNK = 16
N_CHUNK = TOK_PER_W // CHUNK
NBUF = 3
SLICES_PER_ROW = HIDDEN // LANES

_mesh = plsc.VectorSubcoreMesh(core_axis_name="c", subcore_axis_name="s")


@functools.partial(
    pl.kernel,
    mesh=_mesh,
    out_type=jax.ShapeDtypeStruct((N_TOK, HIDDEN), jnp.float32),
    scratch_types=[
        pltpu.VMEM((TOK_PER_W,), jnp.int32),
        pltpu.VMEM((TOK_PER_W,), jnp.int32),
        pltpu.VMEM((NBUF, CHUNK, HIDDEN), jnp.float32),
        pltpu.VMEM((NBUF, CHUNK, HIDDEN), jnp.float32),
    ]
    + [pltpu.SemaphoreType.DMA] * (2 * NBUF),
)
def _embed_sum(ids_hbm, pos_hbm, wtab_hbm, ptab_hbm, out_hbm,
               ids_v, pids_v, wbuf, pbuf, *sems):
    gsem = sems[:NBUF]
    ssem = sems[NBUF:]
    wid = lax.axis_index("s") * NC + lax.axis_index("c")
    base = wid * TOK_PER_W
    pltpu.sync_copy(ids_hbm.at[pl.ds(base, TOK_PER_W)], ids_v)
    pltpu.sync_copy(pos_hbm.at[pl.ds(base, TOK_PER_W)], pids_v)

    def start_g(ci):
        b = ci % NBUF
        idx = pl.ds(ci * CHUNK, CHUNK)
        cw = pltpu.async_copy(wtab_hbm.at[ids_v.at[idx]], wbuf.at[b], gsem[b])
        cp = pltpu.async_copy(ptab_hbm.at[pids_v.at[idx]], pbuf.at[b], gsem[b])
        return cw, cp

    def start_st(ci):
        b = ci % NBUF
        return pltpu.async_copy(
            wbuf.at[b], out_hbm.at[pl.ds(base + ci * CHUNK, CHUNK)], ssem[b])

    g_h = {0: start_g(0), 1: start_g(1)}
    st_h = {}
    for ci in range(N_CHUNK):
        b = ci % NBUF
        cw, cp = g_h.pop(ci)
        cw.wait()
        cp.wait()

        def row_body(r, carry, _b=b):
            for j in range(SLICES_PER_ROW):
                sl = pl.ds(j * LANES, LANES)
                plsc.addupdate(wbuf.at[_b, r, sl], pbuf[_b, r, sl])
            return carry

        lax.fori_loop(0, CHUNK, row_body, 0)

        if ci + 2 < N_CHUNK:
            if ci - 1 >= 0:
                st_h.pop(ci - 1).wait()
            g_h[ci + 2] = start_g(ci + 2)
        st_h[ci] = start_st(ci)
    for ci in sorted(st_h):
        st_h.pop(ci).wait()


def kernel(input_ids, position_ids, word_embeddings, position_embeddings):
    ids = input_ids.reshape(-1).astype(jnp.int32)
    pos = position_ids.reshape(-1).astype(jnp.int32)
    out = _embed_sum(ids, pos, word_embeddings, position_embeddings)
    return out.reshape(input_ids.shape[0], input_ids.shape[1], HIDDEN)
